# baseline (speedup 1.0000x reference)
import functools

import jax
import jax.numpy as jnp
from jax import lax
from jax.experimental import pallas as pl
from jax.experimental.pallas import tpu as pltpu
from jax.experimental.pallas import tpu_sc as plsc

_NC, _NS, _L = 2, 16, 16
_NW = _NC * _NS

_LOG1P_COEFS = (
    -0.01720806024968624,
    0.0817268118262291,
    -0.1887826770544052,
    0.31459054350852966,
    -0.49697792530059814,
    0.9997924566268921,
    3.50755203726294e-06,
)


def _softplus(x):
    t = jnp.exp(-jnp.abs(x))
    p = jnp.full(x.shape, _LOG1P_COEFS[0], jnp.float32)
    for c in _LOG1P_COEFS[1:]:
        p = p * t + c
    return jnp.maximum(x, 0.0) + p


def kernel(user_id, question_id, theta_table, a_table, b_table):
    B = user_id.shape[0]
    D = theta_table.shape[1]
    assert B % (_NW * _L) == 0
    bw = B // _NW
    ng = bw // _L

    mesh = plsc.VectorSubcoreMesh(
        core_axis_name="c", subcore_axis_name="s",
        num_cores=_NC, num_subcores=_NS)

    @functools.partial(
        pl.kernel,
        out_type=jax.ShapeDtypeStruct((B,), jnp.float32),
        mesh=mesh,
        scratch_types=[
            pltpu.VMEM((bw,), jnp.int32),
            pltpu.VMEM((bw,), jnp.int32),
            pltpu.VMEM((D, bw), jnp.float32),
            pltpu.VMEM((D, bw), jnp.float32),
            pltpu.VMEM((bw,), jnp.float32),
            pltpu.VMEM((bw,), jnp.float32),
            pltpu.SemaphoreType.DMA,
            pltpu.SemaphoreType.DMA,
            pltpu.SemaphoreType.DMA,
        ],
        compiler_params=pltpu.CompilerParams(use_tc_tiling_on_sc=False),
    )
    def sc_kernel(uid_hbm, qid_hbm, tht_hbm, at_hbm, b_hbm, out_hbm,
                  uid_v, qid_v, th_t, a_t, b_v, out_v, sem_th, sem_a, sem_b):
        wid = lax.axis_index("s") * _NC + lax.axis_index("c")
        base = wid * bw
        pltpu.sync_copy(uid_hbm.at[pl.ds(base, bw)], uid_v)
        pltpu.sync_copy(qid_hbm.at[pl.ds(base, bw)], qid_v)
        cp_b = pltpu.async_copy(b_hbm.at[qid_v], b_v, sem_b)
        cps = []
        for d in range(D):
            cps.append(pltpu.async_copy(
                tht_hbm.at[d].at[uid_v], th_t.at[d], sem_th))
            cps.append(pltpu.async_copy(
                at_hbm.at[d].at[qid_v], a_t.at[d], sem_a))
        cp_b.wait()
        for cp in cps:
            cp.wait()

        def group(g, carry):
            sl = pl.ds(g * _L, _L)
            acc = -b_v[sl]
            for d in range(D):
                acc = acc + _softplus(a_t[d, sl]) * th_t[d, sl]
            out_v[sl] = 1.0 / (1.0 + jnp.exp(-acc))
            return carry

        lax.fori_loop(0, ng, group, 0)
        pltpu.sync_copy(out_v, out_hbm.at[pl.ds(base, bw)])

    return sc_kernel(user_id, question_id, theta_table.T, a_table.T,
                     b_table.reshape(-1))


# speedup vs baseline: 5.6937x; 5.6937x over previous
import functools

import jax
import jax.numpy as jnp
from jax import lax
from jax.experimental import pallas as pl
from jax.experimental.pallas import tpu as pltpu
from jax.experimental.pallas import tpu_sc as plsc

_NC, _NS, _L = 2, 16, 16
_NW = _NC * _NS
_SR = 128
_CH = 128

_LOG1P_COEFS = (
    -0.01720806024968624,
    0.0817268118262291,
    -0.1887826770544052,
    0.31459054350852966,
    -0.49697792530059814,
    0.9997924566268921,
    3.50755203726294e-06,
)

_GATHER_DNUMS = lax.GatherDimensionNumbers(
    offset_dims=(), collapsed_slice_dims=(0,), start_index_map=(0,))


def _perm(x, idx):
    return lax.gather(x, idx[:, None], dimension_numbers=_GATHER_DNUMS,
                      slice_sizes=(1,),
                      mode=lax.GatherScatterMode.PROMISE_IN_BOUNDS)


def _softplus(x):
    t = jnp.exp(-jnp.abs(x))
    p = jnp.full(x.shape, _LOG1P_COEFS[0], jnp.float32)
    for c in _LOG1P_COEFS[1:]:
        p = p * t + c
    return jnp.maximum(x, 0.0) + p


def _pick_row(ref, el, jb, half):
    v = ref[el, pl.ds(3 * 32 + half, _L)]
    for j in (2, 1, 0):
        v = jnp.where(jb == j, ref[el, pl.ds(j * 32 + half, _L)], v)
    return v


def kernel(user_id, question_id, theta_table, a_table, b_table):
    B = user_id.shape[0]
    D = theta_table.shape[1]
    V = theta_table.shape[0]
    assert D == 32 and B % (_NW * _L) == 0 and (V * D) % _SR == 0
    bw = B // _NW
    nch = bw // _CH
    ngc = _CH // _L

    mesh = plsc.VectorSubcoreMesh(
        core_axis_name="c", subcore_axis_name="s",
        num_cores=_NC, num_subcores=_NS)

    @functools.partial(
        pl.kernel,
        out_type=jax.ShapeDtypeStruct((B,), jnp.float32),
        mesh=mesh,
        scratch_types=[
            pltpu.VMEM((bw,), jnp.int32),
            pltpu.VMEM((bw,), jnp.int32),
            pltpu.VMEM((bw,), jnp.int32),
            pltpu.VMEM((bw,), jnp.int32),
            pltpu.VMEM((2, _CH, _SR), jnp.float32),
            pltpu.VMEM((2, _CH, _SR), jnp.float32),
            pltpu.VMEM((bw,), jnp.float32),
            pltpu.VMEM((bw,), jnp.float32),
            pltpu.SemaphoreType.DMA,
            pltpu.SemaphoreType.DMA,
            pltpu.SemaphoreType.DMA,
        ],
        compiler_params=pltpu.CompilerParams(use_tc_tiling_on_sc=True,
                                             needs_layout_passes=False),
    )
    def sc_kernel(uid_hbm, qid_hbm, th_hbm, a_hbm, b_hbm, out_hbm,
                  uid_v, qid_v, usup_v, qsup_v, th_v, a_v, b_v, out_v,
                  sem_th, sem_a, sem_b):
        wid = lax.axis_index("s") * _NC + lax.axis_index("c")
        base = wid * bw
        pltpu.sync_copy(uid_hbm.at[pl.ds(base, bw)], uid_v)
        pltpu.sync_copy(qid_hbm.at[pl.ds(base, bw)], qid_v)
        cp_b = pltpu.async_copy(b_hbm.at[qid_v], b_v, sem_b)

        def sup(i, carry):
            sl = pl.ds(i * _L, _L)
            usup_v[sl] = lax.shift_right_logical(uid_v[sl], 2)
            qsup_v[sl] = lax.shift_right_logical(qid_v[sl], 2)
            return carry

        lax.fori_loop(0, bw // _L, sup, 0)

        def fire(c, buf):
            sl = pl.ds(c * _CH, _CH)
            cpt = pltpu.async_copy(th_hbm.at[usup_v.at[sl]], th_v.at[buf],
                                   sem_th)
            cpa = pltpu.async_copy(a_hbm.at[qsup_v.at[sl]], a_v.at[buf],
                                   sem_a)
            return cpt, cpa

        lanes = lax.iota(jnp.int32, _L)
        pend = fire(0, 0)
        cp_b.wait()

        for c in range(nch):
            buf = c % 2
            pend[0].wait()
            pend[1].wait()
            if c + 1 < nch:
                pend = fire(c + 1, 1 - buf)

            def group(g, carry):
                sl = pl.ds(c * _CH + g * _L, _L)
                jb16 = jnp.bitwise_and(qid_v[sl], 3)
                ju16 = jnp.bitwise_and(uid_v[sl], 3)
                res = jnp.zeros((_L,), jnp.float32)
                for i in range(_L):
                    el = g * _L + i
                    spl = jnp.full((_L,), i, jnp.int32)
                    ju = _perm(ju16, spl)
                    jq = _perm(jb16, spl)
                    t0 = _pick_row(th_v.at[buf], el, ju, 0)
                    t1 = _pick_row(th_v.at[buf], el, ju, _L)
                    a0 = _pick_row(a_v.at[buf], el, jq, 0)
                    a1 = _pick_row(a_v.at[buf], el, jq, _L)
                    p = _softplus(a0) * t0 + _softplus(a1) * t1
                    for s in (1, 2, 4, 8):
                        p = p + _perm(p, lanes ^ s)
                    res = jnp.where(lanes == i, p, res)
                z = res - b_v[sl]
                out_v[pl.ds(c * _CH + g * _L, _L)] = 1.0 / (1.0 + jnp.exp(-z))
                return carry

            lax.fori_loop(0, ngc, group, 0)

        pltpu.sync_copy(out_v, out_hbm.at[pl.ds(base, bw)])

    return sc_kernel(user_id, question_id,
                     theta_table.reshape(-1, _SR), a_table.reshape(-1, _SR),
                     b_table.reshape(-1))
